# baseline (speedup 1.0000x reference)
import functools

import jax
import jax.numpy as jnp
from jax import lax
from jax.experimental import pallas as pl
from jax.experimental.pallas import tpu as pltpu
from jax.experimental.pallas import tpu_sc as plsc

WORD_SIZE = 100000
EMBED = 64
BATCH = 16384

NUM_CORES = 2
NUM_SUBCORES = 16
NUM_WORKERS = NUM_CORES * NUM_SUBCORES
FEATS_PER_W = EMBED // NUM_WORKERS
LANES = 16
OUT_CHUNK = 4096
N_CHUNKS = BATCH // OUT_CHUNK

HALF = 49920
TAIL_OFF = 2 * HALF
TAIL = WORD_SIZE - TAIL_OFF

_mesh = plsc.VectorSubcoreMesh(core_axis_name="c", subcore_axis_name="s")


@functools.partial(
    pl.kernel,
    mesh=_mesh,
    out_type=jax.ShapeDtypeStruct((EMBED, BATCH), jnp.float32),
    scratch_types=[
        pltpu.VMEM((WORD_SIZE,), jnp.float32),
        pltpu.VMEM((BATCH,), jnp.int32),
        pltpu.VMEM((OUT_CHUNK,), jnp.float32),
        pltpu.VMEM((OUT_CHUNK,), jnp.float32),
        pltpu.SemaphoreType.DMA,
        pltpu.SemaphoreType.DMA,
        pltpu.SemaphoreType.DMA,
        pltpu.SemaphoreType.DMA,
        pltpu.SemaphoreType.DMA,
        pltpu.SemaphoreType.DMA,
    ],
    compiler_params=pltpu.CompilerParams(
        use_tc_tiling_on_sc=True, needs_layout_passes=False
    ),
)
def _embed_gather(
    tab_t_hbm,
    idx_hbm,
    tail_hbm,
    out_t_hbm,
    row_v,
    idx_v,
    out_a,
    out_b,
    sem_i,
    sem_r0,
    sem_r1,
    sem_r2,
    sem_a,
    sem_b,
):
    wid = lax.axis_index("s") * NUM_CORES + lax.axis_index("c")
    f0 = wid * FEATS_PER_W
    out_bufs = [out_a, out_b]
    out_sems = [sem_a, sem_b]

    def fetch_row(feat):
        return [
            pltpu.async_copy(
                tab_t_hbm.at[feat, pl.ds(0, HALF)], row_v.at[pl.ds(0, HALF)], sem_r0
            ),
            pltpu.async_copy(
                tab_t_hbm.at[feat, pl.ds(HALF, HALF)],
                row_v.at[pl.ds(HALF, HALF)],
                sem_r1,
            ),
            pltpu.async_copy(
                tail_hbm.at[pl.ds(feat * TAIL, TAIL)],
                row_v.at[pl.ds(TAIL_OFF, TAIL)],
                sem_r2,
            ),
        ]

    idx_cp = pltpu.async_copy(idx_hbm, idx_v, sem_i)
    row_cps = fetch_row(f0)
    idx_cp.wait()

    flushes = {}
    for f in range(FEATS_PER_W):
        for cp in row_cps:
            cp.wait()
        for c in range(N_CHUNKS):
            buf = out_bufs[c % 2]
            base = c * OUT_CHUNK
            prior = flushes.pop(c % 2, None)
            if prior is not None:
                prior.wait()

            @plsc.parallel_loop(0, OUT_CHUNK, step=LANES, unroll=8)
            def _body(j):
                iv = idx_v[pl.ds(base + j, LANES)]
                buf[pl.ds(j, LANES)] = plsc.load_gather(row_v, [iv])

            if f + 1 < FEATS_PER_W and c == N_CHUNKS - 1:
                row_cps = fetch_row(f0 + f + 1)
            flushes[c % 2] = pltpu.async_copy(
                buf, out_t_hbm.at[f0 + f, pl.ds(base, OUT_CHUNK)], out_sems[c % 2]
            )
    for cp in flushes.values():
        cp.wait()


def kernel(inputs, table):
    idx = inputs.reshape(BATCH).astype(jnp.int32)
    tab_t = table.T
    tail = tab_t[:, TAIL_OFF:].reshape(-1)
    out_t = _embed_gather(tab_t, idx, tail)
    return out_t.T


# speedup vs baseline: 1.0267x; 1.0267x over previous
import functools

import jax
import jax.numpy as jnp
from jax import lax
from jax.experimental import pallas as pl
from jax.experimental.pallas import tpu as pltpu
from jax.experimental.pallas import tpu_sc as plsc

WORD_SIZE = 100000
EMBED = 64
BATCH = 16384

NUM_CORES = 2
NUM_SUBCORES = 16
NUM_WORKERS = NUM_CORES * NUM_SUBCORES
FEATS_PER_W = EMBED // NUM_WORKERS
LANES = 16
OUT_CHUNK = 4096
N_CHUNKS = BATCH // OUT_CHUNK

_mesh = plsc.VectorSubcoreMesh(core_axis_name="c", subcore_axis_name="s")


@functools.partial(
    pl.kernel,
    mesh=_mesh,
    out_type=jax.ShapeDtypeStruct((EMBED, BATCH), jnp.float32),
    scratch_types=[
        pltpu.VMEM((WORD_SIZE,), jnp.float32),
        pltpu.VMEM((BATCH,), jnp.int32),
        pltpu.VMEM((OUT_CHUNK,), jnp.float32),
        pltpu.VMEM((OUT_CHUNK,), jnp.float32),
        pltpu.SemaphoreType.DMA,
        pltpu.SemaphoreType.DMA,
        pltpu.SemaphoreType.DMA,
        pltpu.SemaphoreType.DMA,
    ],
    compiler_params=pltpu.CompilerParams(
        use_tc_tiling_on_sc=True, needs_layout_passes=False
    ),
)
def _embed_gather(
    tab_t_hbm, idx_hbm, out_t_hbm, row_v, idx_v, out_a, out_b, sem_i, sem_r, sem_a, sem_b
):
    wid = lax.axis_index("s") * NUM_CORES + lax.axis_index("c")
    f0 = wid * FEATS_PER_W
    out_bufs = [out_a, out_b]
    out_sems = [sem_a, sem_b]

    idx_cp = pltpu.async_copy(idx_hbm, idx_v, sem_i)
    row_cp = pltpu.async_copy(tab_t_hbm.at[f0], row_v, sem_r)
    idx_cp.wait()

    flushes = {}
    for f in range(FEATS_PER_W):
        row_cp.wait()
        for c in range(N_CHUNKS):
            buf = out_bufs[c % 2]
            base = c * OUT_CHUNK
            prior = flushes.pop(c % 2, None)
            if prior is not None:
                prior.wait()

            @plsc.parallel_loop(0, OUT_CHUNK, step=LANES, unroll=8)
            def _body(j):
                iv = idx_v[pl.ds(base + j, LANES)]
                buf[pl.ds(j, LANES)] = plsc.load_gather(row_v, [iv])

            if f + 1 < FEATS_PER_W and c == N_CHUNKS - 1:
                row_cp = pltpu.async_copy(tab_t_hbm.at[f0 + f + 1], row_v, sem_r)
            flushes[c % 2] = pltpu.async_copy(
                buf, out_t_hbm.at[f0 + f, pl.ds(base, OUT_CHUNK)], out_sems[c % 2]
            )
    for cp in flushes.values():
        cp.wait()


def kernel(inputs, table):
    idx = inputs.reshape(BATCH).astype(jnp.int32)
    out_t = _embed_gather(table.T, idx)
    return out_t.T
